# baseline (speedup 1.0000x reference)
import jax
import jax.numpy as jnp
from jax import lax
from jax.experimental import pallas as pl
from jax.experimental.pallas import tpu as pltpu
from jax.experimental.pallas import tpu_sc as plsc

N_NODES = 10000
N_EDGES = 320000
D = 128

NC = 2
NS = 16
NW = NC * NS
CHUNK = 80
NCHUNK = 128
EPW = CHUNK * NCHUNK
E_PAD = NW * EPW
N_ACC = 10240
ROWS_PER_TILE = N_ACC // NS


def _spmm_body(h_hbm, row_hbm, col_hbm, val_hbm, out_hbm,
               cbuf, rowb, valb, gb0, gb1, gb2, gb3, acc,
               semg0, semg1, semg2, semg3, sems0, sems1, sems2, sems3,
               semr0, semr1, semr2, semr3, semv0, semv1, semv2, semv3,
               semc0, semc1, semc2, semc3):
    c = lax.axis_index("c")
    s = lax.axis_index("s")
    wid = c * NS + s
    gbufs = (gb0, gb1, gb2, gb3)
    semg = (semg0, semg1, semg2, semg3)
    sems = (sems0, sems1, sems2, sems3)
    semr = (semr0, semr1, semr2, semr3)
    semv = (semv0, semv1, semv2, semv3)
    semc = (semc0, semc1, semc2, semc3)

    zero16 = jnp.zeros((16,), jnp.float32)

    def zrow(i, carry):
        for q in range(8):
            gb0[i, pl.ds(16 * q, 16)] = zero16
        return carry

    lax.fori_loop(0, CHUNK, zrow, 0)
    base = s * ROWS_PER_TILE
    for k in range(ROWS_PER_TILE // CHUNK):
        pltpu.sync_copy(gb0, acc.at[pl.ds(base + k * CHUNK, CHUNK)])
    plsc.subcore_barrier()

    def start(j, p, pr):
        pltpu.async_copy(h_hbm.at[cbuf.at[p]], gbufs[p], semg[p])
        pltpu.async_copy(row_hbm.at[wid, j], rowb.at[pr], semr[p])
        pltpu.async_copy(val_hbm.at[wid, j], valb.at[p], semv[p])

    def scale_chunk(p):
        buf = gbufs[p]

        def group_body(g, carry2):
            vv = valb[p, pl.ds(16 * g, 16)]
            for i in range(16):
                vbc = lax.gather(
                    vv, jnp.full((16, 1), i, jnp.int32),
                    dimension_numbers=lax.GatherDimensionNumbers(
                        offset_dims=(), collapsed_slice_dims=(0,),
                        start_index_map=(0,)),
                    slice_sizes=(1,),
                    mode=lax.GatherScatterMode.PROMISE_IN_BOUNDS)
                e = g * 16 + i
                for q in range(8):
                    sl = pl.ds(16 * q, 16)
                    buf[e, sl] = buf[e, sl] * vbc
            return carry2

        lax.fori_loop(0, CHUNK // 16, group_body, 0)

    pltpu.sync_copy(col_hbm.at[wid, 0], cbuf.at[0])
    pltpu.sync_copy(col_hbm.at[wid, 1], cbuf.at[1])
    pltpu.sync_copy(col_hbm.at[wid, 2], cbuf.at[2])
    pltpu.async_copy(col_hbm.at[wid, 3], cbuf.at[3], semc[3])
    start(0, 0, 0)
    start(1, 1, 1)
    start(2, 2, 2)

    def eight_body(ii, carry):
        j0 = 8 * ii
        for off in range(8):
            j = j0 + off
            p = off % 4
            pm = (off + 3) % 4

            pltpu.make_async_copy(h_hbm.at[cbuf.at[p]], gbufs[p],
                                  semg[p]).wait()
            pltpu.make_async_copy(row_hbm.at[wid, j], rowb.at[off],
                                  semr[p]).wait()
            pltpu.make_async_copy(val_hbm.at[wid, j], valb.at[p],
                                  semv[p]).wait()

            @pl.when(j + 4 < NCHUNK)
            def _():
                pltpu.async_copy(col_hbm.at[wid, j + 4], cbuf.at[p], semc[p])

            scale_chunk(p)

            @pl.when(j >= 1)
            def _():
                pltpu.make_async_copy(gbufs[pm],
                                      acc.at[rowb.at[(off + 7) % 8]],
                                      sems[pm]).wait()

            @pl.when(j + 3 < NCHUNK)
            def _():
                pltpu.make_async_copy(col_hbm.at[wid, j + 3], cbuf.at[pm],
                                      semc[pm]).wait()
                start(j + 3, pm, (off + 3) % 8)

            pltpu.async_copy(gbufs[p], acc.at[rowb.at[off]], sems[p],
                             add=True)
        return carry

    lax.fori_loop(0, NCHUNK // 8, eight_body, 0)
    pltpu.make_async_copy(gbufs[3], acc.at[rowb.at[7]], sems[3]).wait()
    plsc.subcore_barrier()

    pltpu.sync_copy(acc.at[pl.ds(base, ROWS_PER_TILE)],
                    out_hbm.at[c, pl.ds(base, ROWS_PER_TILE)])


_spmm = pl.kernel(
    _spmm_body,
    out_type=jax.ShapeDtypeStruct((NC, N_ACC, D), jnp.float32),
    mesh=plsc.VectorSubcoreMesh(core_axis_name="c", subcore_axis_name="s"),
    scratch_types=[
        pltpu.VMEM((4, CHUNK), jnp.int32),
        pltpu.VMEM((8, CHUNK), jnp.int32),
        pltpu.VMEM((4, CHUNK), jnp.float32),
        pltpu.VMEM((CHUNK, D), jnp.float32),
        pltpu.VMEM((CHUNK, D), jnp.float32),
        pltpu.VMEM((CHUNK, D), jnp.float32),
        pltpu.VMEM((CHUNK, D), jnp.float32),
        pltpu.VMEM_SHARED((N_ACC, D), jnp.float32),
    ] + [pltpu.SemaphoreType.DMA] * 20,
)


_BLK = 1000


def _mm_body(x_ref, w_ref, o_ref):
    o_ref[...] = lax.dot_general(
        x_ref[...], w_ref[...], (((1,), (1,)), ((), ())),
        preferred_element_type=jnp.float32)


def _linear(x, w):
    return pl.pallas_call(
        _mm_body,
        grid=(N_NODES // _BLK,),
        in_specs=[pl.BlockSpec((_BLK, D), lambda i: (i, 0)),
                  pl.BlockSpec((D, D), lambda i: (0, 0))],
        out_specs=pl.BlockSpec((_BLK, D), lambda i: (i, 0)),
        out_shape=jax.ShapeDtypeStruct((N_NODES, D), jnp.float32),
    )(x, w)


def _fuse_body(p_ref, w_ref, o_ref):
    h = jnp.maximum(p_ref[0] + p_ref[1], 0.0)
    o_ref[...] = lax.dot_general(
        h, w_ref[...], (((1,), (1,)), ((), ())),
        preferred_element_type=jnp.float32)


def _relu_sum_linear(p, w):
    return pl.pallas_call(
        _fuse_body,
        grid=(N_NODES // _BLK,),
        in_specs=[pl.BlockSpec((NC, _BLK, D), lambda i: (0, i, 0)),
                  pl.BlockSpec((D, D), lambda i: (0, 0))],
        out_specs=pl.BlockSpec((_BLK, D), lambda i: (i, 0)),
        out_shape=jax.ShapeDtypeStruct((N_NODES, D), jnp.float32),
    )(p, w)


def _add_body(p_ref, o_ref):
    o_ref[...] = p_ref[0] + p_ref[1]


def _partial_sum(p):
    return pl.pallas_call(
        _add_body,
        grid=(N_NODES // _BLK,),
        in_specs=[pl.BlockSpec((NC, _BLK, D), lambda i: (0, i, 0))],
        out_specs=pl.BlockSpec((_BLK, D), lambda i: (i, 0)),
        out_shape=jax.ShapeDtypeStruct((N_NODES, D), jnp.float32),
    )(p)


def kernel(x, adj_indices, adj_values, W1, W2):
    row = adj_indices[0].astype(jnp.int32)
    col = adj_indices[1].astype(jnp.int32)
    val = adj_values.astype(jnp.float32)
    pad = E_PAD - N_EDGES
    row3 = jnp.concatenate([row, jnp.zeros((pad,), jnp.int32)]
                           ).reshape(NW, NCHUNK, CHUNK)
    col3 = jnp.concatenate([col, jnp.zeros((pad,), jnp.int32)]
                           ).reshape(NW, NCHUNK, CHUNK)
    val3 = jnp.concatenate([val, jnp.zeros((pad,), jnp.float32)]
                           ).reshape(NW, NCHUNK, CHUNK)

    h1 = _linear(x, W1)
    p = _spmm(h1, row3, col3, val3)
    h2 = _relu_sum_linear(p, W2)
    q = _spmm(h2, row3, col3, val3)
    return _partial_sum(q)


# speedup vs baseline: 1.9642x; 1.9642x over previous
import jax
import jax.numpy as jnp
from jax import lax
from jax.experimental import pallas as pl
from jax.experimental.pallas import tpu as pltpu
from jax.experimental.pallas import tpu_sc as plsc

N_NODES = 10000
N_EDGES = 320000
D = 128

NC = 2
NS = 16
NW = NC * NS
CHUNK = 80
NCHUNK = 126
EPW = CHUNK * NCHUNK
E_PAD = NW * EPW
N_ACC = 10240
ROWS_PER_TILE = N_ACC // NS


def _spmm_body(h_hbm, row_hbm, col_hbm, val_hbm, out_hbm,
               col_v, rowb, valb, gbuf0, gbuf1, gbuf2, acc,
               semg0, semg1, semg2, sems0, sems1, sems2,
               semr0, semr1, semr2, semv0, semv1, semv2):
    c = lax.axis_index("c")
    s = lax.axis_index("s")
    wid = c * NS + s
    gbufs = (gbuf0, gbuf1, gbuf2)
    semg = (semg0, semg1, semg2)
    sems = (sems0, sems1, sems2)
    semr = (semr0, semr1, semr2)
    semv = (semv0, semv1, semv2)

    zero16 = jnp.zeros((16,), jnp.float32)

    def zrow(i, carry):
        for q in range(8):
            gbuf0[i, pl.ds(16 * q, 16)] = zero16
        return carry

    lax.fori_loop(0, CHUNK, zrow, 0)
    base = s * ROWS_PER_TILE
    for k in range(ROWS_PER_TILE // CHUNK):
        pltpu.sync_copy(gbuf0, acc.at[pl.ds(base + k * CHUNK, CHUNK)])
    plsc.subcore_barrier()

    pltpu.sync_copy(col_hbm.at[wid], col_v)

    def start(j, p):
        pltpu.async_copy(h_hbm.at[col_v.at[j]], gbufs[p], semg[p])
        pltpu.async_copy(row_hbm.at[wid, j], rowb.at[p], semr[p])
        pltpu.async_copy(val_hbm.at[wid, j], valb.at[p], semv[p])

    def scale_chunk(p):
        buf = gbufs[p]

        def group_body(g, carry2):
            vv = valb[p, pl.ds(16 * g, 16)]
            for i in range(16):
                vbc = lax.gather(
                    vv, jnp.full((16, 1), i, jnp.int32),
                    dimension_numbers=lax.GatherDimensionNumbers(
                        offset_dims=(), collapsed_slice_dims=(0,),
                        start_index_map=(0,)),
                    slice_sizes=(1,),
                    mode=lax.GatherScatterMode.PROMISE_IN_BOUNDS)
                e = g * 16 + i
                for q in range(8):
                    sl = pl.ds(16 * q, 16)
                    buf[e, sl] = buf[e, sl] * vbc
            return carry2

        lax.fori_loop(0, CHUNK // 16, group_body, 0)

    start(0, 0)
    start(1, 1)

    def ring_body(ii, carry):
        j0 = 3 * ii
        for off in range(3):
            p = off
            pm = (off + 2) % 3
            j = j0 + off

            pltpu.make_async_copy(h_hbm.at[col_v.at[j]], gbufs[p],
                                  semg[p]).wait()
            pltpu.make_async_copy(row_hbm.at[wid, j], rowb.at[p],
                                  semr[p]).wait()
            pltpu.make_async_copy(val_hbm.at[wid, j], valb.at[p],
                                  semv[p]).wait()
            scale_chunk(p)

            @pl.when(j >= 1)
            def _():
                pltpu.make_async_copy(gbufs[pm], acc.at[rowb.at[pm]],
                                      sems[pm]).wait()

            @pl.when(j + 2 < NCHUNK)
            def _():
                start(j + 2, pm)

            pltpu.async_copy(gbufs[p], acc.at[rowb.at[p]], sems[p], add=True)
        return carry

    lax.fori_loop(0, NCHUNK // 3, ring_body, 0)
    pltpu.make_async_copy(gbufs[2], acc.at[rowb.at[2]], sems[2]).wait()
    plsc.subcore_barrier()

    pltpu.sync_copy(acc.at[pl.ds(base, ROWS_PER_TILE)],
                    out_hbm.at[c, pl.ds(base, ROWS_PER_TILE)])


_spmm = pl.kernel(
    _spmm_body,
    out_type=jax.ShapeDtypeStruct((NC, N_ACC, D), jnp.float32),
    mesh=plsc.VectorSubcoreMesh(core_axis_name="c", subcore_axis_name="s"),
    scratch_types=[
        pltpu.VMEM((NCHUNK, CHUNK), jnp.int32),
        pltpu.VMEM((3, CHUNK), jnp.int32),
        pltpu.VMEM((3, CHUNK), jnp.float32),
        pltpu.VMEM((CHUNK, D), jnp.float32),
        pltpu.VMEM((CHUNK, D), jnp.float32),
        pltpu.VMEM((CHUNK, D), jnp.float32),
        pltpu.VMEM_SHARED((N_ACC, D), jnp.float32),
    ] + [pltpu.SemaphoreType.DMA] * 12,
)


_BLK = 1000


def _mm_body(x_ref, w_ref, o_ref):
    o_ref[...] = lax.dot_general(
        x_ref[...], w_ref[...], (((1,), (1,)), ((), ())),
        preferred_element_type=jnp.float32)


def _linear(x, w):
    return pl.pallas_call(
        _mm_body,
        grid=(N_NODES // _BLK,),
        in_specs=[pl.BlockSpec((_BLK, D), lambda i: (i, 0)),
                  pl.BlockSpec((D, D), lambda i: (0, 0))],
        out_specs=pl.BlockSpec((_BLK, D), lambda i: (i, 0)),
        out_shape=jax.ShapeDtypeStruct((N_NODES, D), jnp.float32),
    )(x, w)


def _fuse_body(p_ref, w_ref, o_ref):
    h = jnp.maximum(p_ref[0] + p_ref[1], 0.0)
    o_ref[...] = lax.dot_general(
        h, w_ref[...], (((1,), (1,)), ((), ())),
        preferred_element_type=jnp.float32)


def _relu_sum_linear(p, w):
    return pl.pallas_call(
        _fuse_body,
        grid=(N_NODES // _BLK,),
        in_specs=[pl.BlockSpec((NC, _BLK, D), lambda i: (0, i, 0)),
                  pl.BlockSpec((D, D), lambda i: (0, 0))],
        out_specs=pl.BlockSpec((_BLK, D), lambda i: (i, 0)),
        out_shape=jax.ShapeDtypeStruct((N_NODES, D), jnp.float32),
    )(p, w)


def _add_body(p_ref, o_ref):
    o_ref[...] = p_ref[0] + p_ref[1]


def _partial_sum(p):
    return pl.pallas_call(
        _add_body,
        grid=(N_NODES // _BLK,),
        in_specs=[pl.BlockSpec((NC, _BLK, D), lambda i: (0, i, 0))],
        out_specs=pl.BlockSpec((_BLK, D), lambda i: (i, 0)),
        out_shape=jax.ShapeDtypeStruct((N_NODES, D), jnp.float32),
    )(p)


def kernel(x, adj_indices, adj_values, W1, W2):
    row = adj_indices[0].astype(jnp.int32)
    col = adj_indices[1].astype(jnp.int32)
    val = adj_values.astype(jnp.float32)
    pad = E_PAD - N_EDGES
    row3 = jnp.concatenate([row, jnp.zeros((pad,), jnp.int32)]
                           ).reshape(NW, NCHUNK, CHUNK)
    col3 = jnp.concatenate([col, jnp.zeros((pad,), jnp.int32)]
                           ).reshape(NW, NCHUNK, CHUNK)
    val3 = jnp.concatenate([val, jnp.zeros((pad,), jnp.float32)]
                           ).reshape(NW, NCHUNK, CHUNK)

    h1 = _linear(x, W1)
    p = _spmm(h1, row3, col3, val3)
    h2 = _relu_sum_linear(p, W2)
    q = _spmm(h2, row3, col3, val3)
    return _partial_sum(q)
